# baseline (speedup 1.0000x reference)
import functools

import jax
import jax.numpy as jnp
from jax import lax
from jax.experimental import pallas as pl
from jax.experimental.pallas import tpu as pltpu
from jax.experimental.pallas import tpu_sc as plsc

NSEG = 10_000
NTILES = 32
CHUNK = 2_000
MASK_BLK = 128_000
GATHER_TILES = 25
GROWS = NSEG // GATHER_TILES

_f32 = jnp.float32
_i32 = jnp.int32


def _take(x, idx):
    return x.at[idx].get(mode="promise_in_bounds")


def _mask_kernel(h3_ref, h4_ref, h5_ref, h6_ref, chi_ref, chim_ref):
    h3 = h3_ref[...]
    h4 = h4_ref[...]
    h5 = h5_ref[...]
    h6 = h6_ref[...]
    filt = (h4 > h3) & (h4 >= h5) & (h4 >= h6)
    chim_ref[...] = jnp.where(filt, chi_ref[...], jnp.inf)


def _build_stage_a(nrows):
    rows_per_tile = nrows // NTILES
    nchunks = rows_per_tile // CHUNK
    groups = CHUNK // 16
    mesh = plsc.VectorSubcoreMesh(core_axis_name="c", subcore_axis_name="s")
    out_type = [
        jax.ShapeDtypeStruct((NTILES, NSEG), _f32),
        jax.ShapeDtypeStruct((NTILES, NSEG), _i32),
        jax.ShapeDtypeStruct((NTILES, NSEG), _f32),
        jax.ShapeDtypeStruct((NTILES, NSEG), _f32),
        jax.ShapeDtypeStruct((NTILES, NSEG), _f32),
        jax.ShapeDtypeStruct((NTILES, NSEG), _f32),
    ]
    scratch = (
        [pltpu.VMEM((CHUNK,), _f32) for _ in range(5)]
        + [pltpu.VMEM((CHUNK,), _i32)]
        + [pltpu.VMEM((CHUNK,), _f32) for _ in range(5)]
        + [pltpu.VMEM((CHUNK,), _i32)]
        + [pltpu.VMEM((NSEG,), _f32), pltpu.VMEM((NSEG,), _i32),
           pltpu.VMEM((NSEG,), _f32), pltpu.VMEM((NSEG,), _f32),
           pltpu.VMEM((NSEG,), _f32), pltpu.VMEM((NSEG,), _f32)]
        + [pltpu.SemaphoreType.DMA, pltpu.SemaphoreType.DMA]
    )

    @functools.partial(pl.kernel, mesh=mesh, out_type=out_type,
                       scratch_types=scratch,
                       compiler_params=pltpu.CompilerParams(
                           needs_layout_passes=False))
    def stage_a(cm_hbm, e_hbm, h0_hbm, h1_hbm, h2_hbm, bi_hbm,
                mc_o, mi_o, sx_o, sy_o, sz_o, se_o,
                cm_b0, e_b0, h0_b0, h1_b0, h2_b0, bi_b0,
                cm_b1, e_b1, h1_b1_, h1_b1b, h2_b1, bi_b1,
                mc_v, mi_v, sx_v, sy_v, sz_v, se_v,
                sem0, sem1):
        cid = lax.axis_index("c")
        sid = lax.axis_index("s")
        wid = sid * 2 + cid
        row_base = wid * rows_per_tile

        iota = lax.iota(_i32, 16)
        inf16 = jnp.full((16,), jnp.inf, _f32)
        nrows16 = jnp.full((16,), nrows, _i32)
        zero16 = jnp.zeros((16,), _f32)

        def init_body(i, c):
            sl = pl.ds(i * 16, 16)
            mc_v[sl] = inf16
            mi_v[sl] = nrows16
            sx_v[sl] = zero16
            sy_v[sl] = zero16
            sz_v[sl] = zero16
            se_v[sl] = zero16
            return c

        lax.fori_loop(0, NSEG // 16, init_body, 0)

        hbm_refs = (cm_hbm, e_hbm, h0_hbm, h1_hbm, h2_hbm, bi_hbm)
        bufs = ((cm_b0, e_b0, h0_b0, h1_b0, h2_b0, bi_b0),
                (cm_b1, e_b1, h1_b1_, h1_b1b, h2_b1, bi_b1))
        sems = (sem0, sem1)

        def start(chunk, b):
            r0 = row_base + chunk * CHUNK
            for src, dst in zip(hbm_refs, bufs[b]):
                pltpu.async_copy(src.at[pl.ds(r0, CHUNK)], dst, sems[b])

        def wait(b):
            for src, dst in zip(hbm_refs, bufs[b]):
                pltpu.make_async_copy(src.at[pl.ds(0, CHUNK)], dst,
                                      sems[b]).wait()

        def compute(b, chunk):
            cm_r, e_r, h0_r, h1_r, h2_r, bi_r = bufs[b]
            row0 = row_base + chunk * CHUNK

            def group(g, carry):
                ro = g * 16
                sl = pl.ds(ro, 16)
                seg = bi_r[sl]
                s_lo = jnp.min(seg)
                s_hi = jnp.max(seg)
                cs = cm_r[sl]
                ev = e_r[sl]
                a0 = h0_r[sl]
                a1 = h1_r[sl]
                a2 = h2_r[sl]
                ix = jnp.where(cs < inf16, (row0 + ro) + iota, nrows16)

                lane0 = iota == 0

                @pl.when(s_lo == s_hi)
                def _():
                    rmin = jnp.min(cs)
                    rix = jnp.min(jnp.where(cs == rmin, ix, nrows16))
                    cur_c = plsc.load_gather(mc_v, [seg])
                    cur_i = plsc.load_gather(mi_v, [seg])
                    better = (rmin < cur_c) | ((rmin == cur_c) & (rix < cur_i))
                    plsc.store_scatter(mc_v, [seg],
                                       jnp.where(better, rmin, cur_c),
                                       mask=lane0)
                    plsc.store_scatter(mi_v, [seg],
                                       jnp.where(better, rix, cur_i),
                                       mask=lane0)
                    plsc.addupdate_scatter(
                        sx_v, [seg], jnp.full((16,), 1.0, _f32) * jnp.sum(a0 * ev),
                        mask=lane0)
                    plsc.addupdate_scatter(
                        sy_v, [seg], jnp.full((16,), 1.0, _f32) * jnp.sum(a1 * ev),
                        mask=lane0)
                    plsc.addupdate_scatter(
                        sz_v, [seg], jnp.full((16,), 1.0, _f32) * jnp.sum(a2 * ev),
                        mask=lane0)
                    plsc.addupdate_scatter(
                        se_v, [seg], jnp.full((16,), 1.0, _f32) * jnp.sum(ev),
                        mask=lane0)

                @pl.when(s_lo != s_hi)
                def _():
                    _slow_group(seg, cs, ix, ev, a0, a1, a2)
                return carry

            def _slow_group(seg, cs, ix, ev, a0, a1, a2):
                for k in (1, 2, 4, 8):
                    pidx = jnp.maximum(iota - k, 0)
                    pseg = _take(seg, pidx)
                    pcs = _take(cs, pidx)
                    pix = _take(ix, pidx)
                    take = (pseg == seg) & (
                        (pcs < cs) | ((pcs == cs) & (pix < ix)))
                    cs = jnp.where(take, pcs, cs)
                    ix = jnp.where(take, pix, ix)

                nseg = _take(seg, jnp.minimum(iota + 1, 15))
                islast = (seg != nseg) | (iota == 15)

                cur_c = plsc.load_gather(mc_v, [seg])
                cur_i = plsc.load_gather(mi_v, [seg])
                better = (cs < cur_c) | ((cs == cur_c) & (ix < cur_i))
                plsc.store_scatter(mc_v, [seg], jnp.where(better, cs, cur_c),
                                   mask=islast)
                plsc.store_scatter(mi_v, [seg], jnp.where(better, ix, cur_i),
                                   mask=islast)

                prev1 = _take(seg, jnp.maximum(iota - 1, 0))
                firstrun = (seg != prev1) | (iota == 0)
                run_start = plsc.cummax(jnp.where(firstrun, iota, 0))

                def run_sum(v):
                    cum = plsc.cumsum(v)
                    base = _take(cum - v, run_start)
                    return cum - base

                plsc.addupdate_scatter(sx_v, [seg], run_sum(a0 * ev),
                                       mask=islast)
                plsc.addupdate_scatter(sy_v, [seg], run_sum(a1 * ev),
                                       mask=islast)
                plsc.addupdate_scatter(sz_v, [seg], run_sum(a2 * ev),
                                       mask=islast)
                plsc.addupdate_scatter(se_v, [seg], run_sum(ev), mask=islast)

            lax.fori_loop(0, groups, group, 0)

        start(0, 0)

        def outer(s, carry):
            for b in range(2):
                chunk = s * 2 + b
                wait(b)

                @pl.when(chunk + 1 < nchunks)
                def _():
                    start(chunk + 1, 1 - b)

                compute(b, chunk)
            return carry

        lax.fori_loop(0, nchunks // 2, outer, 0)

        pltpu.sync_copy(mc_v, mc_o.at[wid])
        pltpu.sync_copy(mi_v, mi_o.at[wid])
        pltpu.sync_copy(sx_v, sx_o.at[wid])
        pltpu.sync_copy(sy_v, sy_o.at[wid])
        pltpu.sync_copy(sz_v, sz_o.at[wid])
        pltpu.sync_copy(se_v, se_o.at[wid])

    return stage_a


def _build_stage_c():
    mesh = plsc.VectorSubcoreMesh(core_axis_name="c", subcore_axis_name="s")
    out_type = [jax.ShapeDtypeStruct((NSEG,), _f32) for _ in range(6)]
    scratch = (
        [pltpu.VMEM((GROWS,), _i32)]
        + [pltpu.VMEM((GROWS,), _f32) for _ in range(6)]
        + [pltpu.SemaphoreType.DMA for _ in range(6)]
    )

    @functools.partial(pl.kernel, mesh=mesh, out_type=out_type,
                       scratch_types=scratch,
                       compiler_params=pltpu.CompilerParams(
                           needs_layout_passes=False,
                           use_tc_tiling_on_sc=False))
    def stage_c(px_hbm, py_hbm, pz_hbm, hx_hbm, hy_hbm, hz_hbm, amin_hbm,
                px_o, py_o, pz_o, hx_o, hy_o, hz_o,
                amin_v, gp0, gp1, gp2, gh0, gh1, gh2,
                s0, s1, s2, s3, s4, s5):
        cid = lax.axis_index("c")
        sid = lax.axis_index("s")
        wid = sid * 2 + cid

        @pl.when(wid < GATHER_TILES)
        def _():
            base = wid * GROWS
            pltpu.sync_copy(amin_hbm.at[pl.ds(base, GROWS)], amin_v)
            cps = [
                pltpu.async_copy(px_hbm.at[amin_v], gp0, s0),
                pltpu.async_copy(py_hbm.at[amin_v], gp1, s1),
                pltpu.async_copy(pz_hbm.at[amin_v], gp2, s2),
                pltpu.async_copy(hx_hbm.at[amin_v], gh0, s3),
                pltpu.async_copy(hy_hbm.at[amin_v], gh1, s4),
                pltpu.async_copy(hz_hbm.at[amin_v], gh2, s5),
            ]
            for cp in cps:
                cp.wait()
            sl = pl.ds(base, GROWS)
            pltpu.sync_copy(gp0, px_o.at[sl])
            pltpu.sync_copy(gp1, py_o.at[sl])
            pltpu.sync_copy(gp2, pz_o.at[sl])
            pltpu.sync_copy(gh0, hx_o.at[sl])
            pltpu.sync_copy(gh1, hy_o.at[sl])
            pltpu.sync_copy(gh2, hz_o.at[sl])

    return stage_c


def _merge_kernel(mc_ref, mi_ref, sx_ref, sy_ref, sz_ref, se_ref,
                  amin_ref, bary_ref, *, nrows):
    bc = mc_ref[0]
    bi = mi_ref[0]
    for k in range(1, NTILES):
        c = mc_ref[k]
        i = mi_ref[k]
        t = (c < bc) | ((c == bc) & (i < bi))
        bc = jnp.where(t, c, bc)
        bi = jnp.where(t, i, bi)
    amin_ref[...] = jnp.minimum(jnp.maximum(bi, 0), nrows - 1)
    sx = jnp.sum(sx_ref[...], axis=0)
    sy = jnp.sum(sy_ref[...], axis=0)
    sz = jnp.sum(sz_ref[...], axis=0)
    se = jnp.sum(se_ref[...], axis=0)
    bary_ref[...] = jnp.stack([sx, sy, sz], axis=1) / se[:, None]


def _final_kernel(px_ref, py_ref, pz_ref, hx_ref, hy_ref, hz_ref, bary_ref,
                  pt_ref, pdir_ref, diff_ref):
    px = px_ref[...]
    py = py_ref[...]
    pz = pz_ref[...]
    pt_ref[...] = jnp.sqrt(px * px + py * py + pz * pz)
    pdir_ref[...] = jnp.stack([px, py, pz], axis=1)
    diff_ref[...] = bary_ref[...] - jnp.stack(
        [hx_ref[...], hy_ref[...], hz_ref[...]], axis=1)


def kernel(x_global_features, h, pos_pxpypz_at_vertex, chi_squared_tracks,
           batch_idx):
    del x_global_features
    nrows = h.shape[0]
    bi = batch_idx.astype(_i32)
    chi = chi_squared_tracks.astype(_f32)

    h0, h1, h2, h3, h4, h5, h6 = (h[:, c] for c in range(7))
    e = h[:, 8]
    px, py, pz = (pos_pxpypz_at_vertex[:, c] for c in range(3))

    grid = nrows // MASK_BLK
    bspec = pl.BlockSpec((MASK_BLK,), lambda i: (i,))
    chi_m = pl.pallas_call(
        _mask_kernel,
        grid=(grid,),
        in_specs=[bspec] * 5,
        out_specs=bspec,
        out_shape=jax.ShapeDtypeStruct((nrows,), _f32),
    )(h3, h4, h5, h6, chi)

    mc, mi, sx, sy, sz, se = _build_stage_a(nrows)(chi_m, e, h0, h1, h2, bi)

    amin, bary = pl.pallas_call(
        functools.partial(_merge_kernel, nrows=nrows),
        out_shape=[jax.ShapeDtypeStruct((NSEG,), _i32),
                   jax.ShapeDtypeStruct((NSEG, 3), _f32)],
    )(mc, mi, sx, sy, sz, se)

    gpx, gpy, gpz, ghx, ghy, ghz = _build_stage_c()(
        px, py, pz, h0, h1, h2, amin)

    ptracks, pdir, diff = pl.pallas_call(
        _final_kernel,
        out_shape=[jax.ShapeDtypeStruct((NSEG,), _f32),
                   jax.ShapeDtypeStruct((NSEG, 3), _f32),
                   jax.ShapeDtypeStruct((NSEG, 3), _f32)],
    )(gpx, gpy, gpz, ghx, ghy, ghz, bary)

    return (ptracks, pdir, diff)


# speedup vs baseline: 1.2793x; 1.2793x over previous
import functools

import jax
import jax.numpy as jnp
from jax import lax
from jax.experimental import pallas as pl
from jax.experimental.pallas import tpu as pltpu
from jax.experimental.pallas import tpu_sc as plsc

NSEG = 10_000
NTILES = 32
CHUNK = 2_000
MASK_BLK = 128_000
GATHER_TILES = 25
GROWS = NSEG // GATHER_TILES

_f32 = jnp.float32
_i32 = jnp.int32


def _take(x, idx):
    return x.at[idx].get(mode="promise_in_bounds")


def _mask_kernel(h3_ref, h4_ref, h5_ref, h6_ref, chi_ref, chim_ref):
    h3 = h3_ref[...]
    h4 = h4_ref[...]
    h5 = h5_ref[...]
    h6 = h6_ref[...]
    filt = (h4 > h3) & (h4 >= h5) & (h4 >= h6)
    chim_ref[...] = jnp.where(filt, chi_ref[...], jnp.inf)


def _build_stage_a(nrows):
    rows_per_tile = nrows // NTILES
    nchunks = rows_per_tile // CHUNK
    groups = CHUNK // 16
    mesh = plsc.VectorSubcoreMesh(core_axis_name="c", subcore_axis_name="s")
    out_type = [
        jax.ShapeDtypeStruct((NTILES, NSEG), _f32),
        jax.ShapeDtypeStruct((NTILES, NSEG), _i32),
        jax.ShapeDtypeStruct((NTILES, NSEG), _f32),
        jax.ShapeDtypeStruct((NTILES, NSEG), _f32),
        jax.ShapeDtypeStruct((NTILES, NSEG), _f32),
        jax.ShapeDtypeStruct((NTILES, NSEG), _f32),
    ]
    scratch = (
        [pltpu.VMEM((CHUNK,), _f32) for _ in range(5)]
        + [pltpu.VMEM((CHUNK,), _i32)]
        + [pltpu.VMEM((CHUNK,), _f32) for _ in range(5)]
        + [pltpu.VMEM((CHUNK,), _i32)]
        + [pltpu.VMEM((NSEG,), _f32), pltpu.VMEM((NSEG,), _i32),
           pltpu.VMEM((NSEG,), _f32), pltpu.VMEM((NSEG,), _f32),
           pltpu.VMEM((NSEG,), _f32), pltpu.VMEM((NSEG,), _f32)]
        + [pltpu.SemaphoreType.DMA, pltpu.SemaphoreType.DMA]
    )

    @functools.partial(pl.kernel, mesh=mesh, out_type=out_type,
                       scratch_types=scratch,
                       compiler_params=pltpu.CompilerParams(
                           needs_layout_passes=False))
    def stage_a(cm_hbm, e_hbm, h0_hbm, h1_hbm, h2_hbm, bi_hbm,
                mc_o, mi_o, sx_o, sy_o, sz_o, se_o,
                cm_b0, e_b0, h0_b0, h1_b0, h2_b0, bi_b0,
                cm_b1, e_b1, h1_b1_, h1_b1b, h2_b1, bi_b1,
                mc_v, mi_v, sx_v, sy_v, sz_v, se_v,
                sem0, sem1):
        cid = lax.axis_index("c")
        sid = lax.axis_index("s")
        wid = sid * 2 + cid
        row_base = wid * rows_per_tile

        iota = lax.iota(_i32, 16)
        inf16 = jnp.full((16,), jnp.inf, _f32)
        nrows16 = jnp.full((16,), nrows, _i32)
        zero16 = jnp.zeros((16,), _f32)

        def init_body(i, c):
            sl = pl.ds(i * 16, 16)
            mc_v[sl] = inf16
            mi_v[sl] = nrows16
            sx_v[sl] = zero16
            sy_v[sl] = zero16
            sz_v[sl] = zero16
            se_v[sl] = zero16
            return c

        lax.fori_loop(0, NSEG // 16, init_body, 0)

        hbm_refs = (cm_hbm, e_hbm, h0_hbm, h1_hbm, h2_hbm, bi_hbm)
        bufs = ((cm_b0, e_b0, h0_b0, h1_b0, h2_b0, bi_b0),
                (cm_b1, e_b1, h1_b1_, h1_b1b, h2_b1, bi_b1))
        sems = (sem0, sem1)

        def start(chunk, b):
            r0 = row_base + chunk * CHUNK
            for src, dst in zip(hbm_refs, bufs[b]):
                pltpu.async_copy(src.at[pl.ds(r0, CHUNK)], dst, sems[b])

        def wait(b):
            for src, dst in zip(hbm_refs, bufs[b]):
                pltpu.make_async_copy(src.at[pl.ds(0, CHUNK)], dst,
                                      sems[b]).wait()

        def compute(b, chunk):
            cm_r, e_r, h0_r, h1_r, h2_r, bi_r = bufs[b]
            row0 = row_base + chunk * CHUNK

            def group(g, carry):
                ro = g * 16
                sl = pl.ds(ro, 16)
                seg = bi_r[sl]
                cs = cm_r[sl]
                ev = e_r[sl]
                a0 = h0_r[sl]
                a1 = h1_r[sl]
                a2 = h2_r[sl]
                ix = jnp.where(cs < inf16, (row0 + ro) + iota, nrows16)

                _scan_group(seg, cs, ix, ev, a0, a1, a2)
                return carry

            def _scan_group(seg, cs, ix, ev, a0, a1, a2):
                for k in (1, 2, 4, 8):
                    pidx = jnp.maximum(iota - k, 0)
                    pseg = _take(seg, pidx)
                    pcs = _take(cs, pidx)
                    pix = _take(ix, pidx)
                    take = (pseg == seg) & (
                        (pcs < cs) | ((pcs == cs) & (pix < ix)))
                    cs = jnp.where(take, pcs, cs)
                    ix = jnp.where(take, pix, ix)

                nseg = _take(seg, jnp.minimum(iota + 1, 15))
                islast = (seg != nseg) | (iota == 15)

                cur_c = plsc.load_gather(mc_v, [seg])
                cur_i = plsc.load_gather(mi_v, [seg])
                better = (cs < cur_c) | ((cs == cur_c) & (ix < cur_i))
                plsc.store_scatter(mc_v, [seg], jnp.where(better, cs, cur_c),
                                   mask=islast)
                plsc.store_scatter(mi_v, [seg], jnp.where(better, ix, cur_i),
                                   mask=islast)

                prev1 = _take(seg, jnp.maximum(iota - 1, 0))
                firstrun = (seg != prev1) | (iota == 0)
                run_start = plsc.cummax(jnp.where(firstrun, iota, 0))

                def run_sum(v):
                    cum = plsc.cumsum(v)
                    base = _take(cum - v, run_start)
                    return cum - base

                plsc.addupdate_scatter(sx_v, [seg], run_sum(a0 * ev),
                                       mask=islast)
                plsc.addupdate_scatter(sy_v, [seg], run_sum(a1 * ev),
                                       mask=islast)
                plsc.addupdate_scatter(sz_v, [seg], run_sum(a2 * ev),
                                       mask=islast)
                plsc.addupdate_scatter(se_v, [seg], run_sum(ev), mask=islast)

            lax.fori_loop(0, groups, group, 0)

        start(0, 0)

        def outer(s, carry):
            for b in range(2):
                chunk = s * 2 + b
                wait(b)

                @pl.when(chunk + 1 < nchunks)
                def _():
                    start(chunk + 1, 1 - b)

                compute(b, chunk)
            return carry

        lax.fori_loop(0, nchunks // 2, outer, 0)

        pltpu.sync_copy(mc_v, mc_o.at[wid])
        pltpu.sync_copy(mi_v, mi_o.at[wid])
        pltpu.sync_copy(sx_v, sx_o.at[wid])
        pltpu.sync_copy(sy_v, sy_o.at[wid])
        pltpu.sync_copy(sz_v, sz_o.at[wid])
        pltpu.sync_copy(se_v, se_o.at[wid])

    return stage_a


def _build_stage_c():
    mesh = plsc.VectorSubcoreMesh(core_axis_name="c", subcore_axis_name="s")
    out_type = [jax.ShapeDtypeStruct((NSEG,), _f32) for _ in range(6)]
    scratch = (
        [pltpu.VMEM((GROWS,), _i32)]
        + [pltpu.VMEM((GROWS,), _f32) for _ in range(6)]
        + [pltpu.SemaphoreType.DMA for _ in range(6)]
    )

    @functools.partial(pl.kernel, mesh=mesh, out_type=out_type,
                       scratch_types=scratch,
                       compiler_params=pltpu.CompilerParams(
                           needs_layout_passes=False,
                           use_tc_tiling_on_sc=False))
    def stage_c(px_hbm, py_hbm, pz_hbm, hx_hbm, hy_hbm, hz_hbm, amin_hbm,
                px_o, py_o, pz_o, hx_o, hy_o, hz_o,
                amin_v, gp0, gp1, gp2, gh0, gh1, gh2,
                s0, s1, s2, s3, s4, s5):
        cid = lax.axis_index("c")
        sid = lax.axis_index("s")
        wid = sid * 2 + cid

        @pl.when(wid < GATHER_TILES)
        def _():
            base = wid * GROWS
            pltpu.sync_copy(amin_hbm.at[pl.ds(base, GROWS)], amin_v)
            cps = [
                pltpu.async_copy(px_hbm.at[amin_v], gp0, s0),
                pltpu.async_copy(py_hbm.at[amin_v], gp1, s1),
                pltpu.async_copy(pz_hbm.at[amin_v], gp2, s2),
                pltpu.async_copy(hx_hbm.at[amin_v], gh0, s3),
                pltpu.async_copy(hy_hbm.at[amin_v], gh1, s4),
                pltpu.async_copy(hz_hbm.at[amin_v], gh2, s5),
            ]
            for cp in cps:
                cp.wait()
            sl = pl.ds(base, GROWS)
            pltpu.sync_copy(gp0, px_o.at[sl])
            pltpu.sync_copy(gp1, py_o.at[sl])
            pltpu.sync_copy(gp2, pz_o.at[sl])
            pltpu.sync_copy(gh0, hx_o.at[sl])
            pltpu.sync_copy(gh1, hy_o.at[sl])
            pltpu.sync_copy(gh2, hz_o.at[sl])

    return stage_c


def _merge_kernel(mc_ref, mi_ref, sx_ref, sy_ref, sz_ref, se_ref,
                  amin_ref, bary_ref, *, nrows):
    bc = mc_ref[0]
    bi = mi_ref[0]
    for k in range(1, NTILES):
        c = mc_ref[k]
        i = mi_ref[k]
        t = (c < bc) | ((c == bc) & (i < bi))
        bc = jnp.where(t, c, bc)
        bi = jnp.where(t, i, bi)
    amin_ref[...] = jnp.minimum(jnp.maximum(bi, 0), nrows - 1)
    sx = jnp.sum(sx_ref[...], axis=0)
    sy = jnp.sum(sy_ref[...], axis=0)
    sz = jnp.sum(sz_ref[...], axis=0)
    se = jnp.sum(se_ref[...], axis=0)
    bary_ref[...] = jnp.stack([sx, sy, sz], axis=1) / se[:, None]


def _final_kernel(px_ref, py_ref, pz_ref, hx_ref, hy_ref, hz_ref, bary_ref,
                  pt_ref, pdir_ref, diff_ref):
    px = px_ref[...]
    py = py_ref[...]
    pz = pz_ref[...]
    pt_ref[...] = jnp.sqrt(px * px + py * py + pz * pz)
    pdir_ref[...] = jnp.stack([px, py, pz], axis=1)
    diff_ref[...] = bary_ref[...] - jnp.stack(
        [hx_ref[...], hy_ref[...], hz_ref[...]], axis=1)


def kernel(x_global_features, h, pos_pxpypz_at_vertex, chi_squared_tracks,
           batch_idx):
    del x_global_features
    nrows = h.shape[0]
    bi = batch_idx.astype(_i32)
    chi = chi_squared_tracks.astype(_f32)

    h0, h1, h2, h3, h4, h5, h6 = (h[:, c] for c in range(7))
    e = h[:, 8]
    px, py, pz = (pos_pxpypz_at_vertex[:, c] for c in range(3))

    grid = nrows // MASK_BLK
    bspec = pl.BlockSpec((MASK_BLK,), lambda i: (i,))
    chi_m = pl.pallas_call(
        _mask_kernel,
        grid=(grid,),
        in_specs=[bspec] * 5,
        out_specs=bspec,
        out_shape=jax.ShapeDtypeStruct((nrows,), _f32),
    )(h3, h4, h5, h6, chi)

    mc, mi, sx, sy, sz, se = _build_stage_a(nrows)(chi_m, e, h0, h1, h2, bi)

    amin, bary = pl.pallas_call(
        functools.partial(_merge_kernel, nrows=nrows),
        out_shape=[jax.ShapeDtypeStruct((NSEG,), _i32),
                   jax.ShapeDtypeStruct((NSEG, 3), _f32)],
    )(mc, mi, sx, sy, sz, se)

    gpx, gpy, gpz, ghx, ghy, ghz = _build_stage_c()(
        px, py, pz, h0, h1, h2, amin)

    ptracks, pdir, diff = pl.pallas_call(
        _final_kernel,
        out_shape=[jax.ShapeDtypeStruct((NSEG,), _f32),
                   jax.ShapeDtypeStruct((NSEG, 3), _f32),
                   jax.ShapeDtypeStruct((NSEG, 3), _f32)],
    )(gpx, gpy, gpz, ghx, ghy, ghz, bary)

    return (ptracks, pdir, diff)
